# Initial kernel scaffold; baseline (speedup 1.0000x reference)
import jax
import jax.numpy as jnp
from jax.experimental import pallas as pl


def kernel(x, edge_index, edge_weight, kernel, bias):
    raise NotImplementedError("write your pallas kernel here")


# speedup vs baseline: 1.3053x; 1.3053x over previous
import functools

import jax
import jax.numpy as jnp
from jax import lax
from jax.experimental import pallas as pl
from jax.experimental.pallas import tpu as pltpu
from jax.experimental.pallas import tpu_sc as plsc

B, M, F, K, E = 8, 10000, 128, 3, 320000
NC, NS = 2, 16
EP = E // NS
G = 80
NCHUNK = EP // G
DR = 624
TAIL = M - NS * DR
ZR = 208
BPC = B // NC


def _spmm_sc(xflat, src, dst, w):
    mesh = plsc.VectorSubcoreMesh(
        core_axis_name="c", subcore_axis_name="s", num_cores=NC, num_subcores=NS
    )

    @functools.partial(
        pl.kernel,
        out_type=jax.ShapeDtypeStruct((B * M, F), jnp.float32),
        mesh=mesh,
        scratch_types=[
            pltpu.VMEM_SHARED((M, F), jnp.float32),
            pltpu.VMEM((ZR, F), jnp.float32),
            pltpu.VMEM((G,), jnp.int32),
            pltpu.VMEM((G,), jnp.int32),
            pltpu.VMEM((G,), jnp.int32),
            pltpu.VMEM((G,), jnp.float32),
            pltpu.VMEM((G, F), jnp.float32),
            pltpu.SemaphoreType.DMA,
        ],
    )
    def run(x_hbm, src_hbm, dst_hbm, w_hbm, y_hbm,
            acc, zbuf, srcv, gidx, dstv, wv, rows, gsem):
        cid = lax.axis_index("c")
        sid = lax.axis_index("s")

        @pl.loop(0, ZR)
        def _zfill(r):
            for j in range(F // 16):
                zbuf[r, pl.ds(j * 16, 16)] = jnp.zeros((16,), jnp.float32)

        for bi in range(BPC):
            b = cid * BPC + bi

            for z in range(DR // ZR):
                pltpu.sync_copy(zbuf, acc.at[pl.ds(sid * DR + z * ZR, ZR)])

            @pl.when(sid == NS - 1)
            def _ztail():
                pltpu.sync_copy(zbuf.at[pl.ds(0, TAIL)], acc.at[pl.ds(NS * DR, TAIL)])

            plsc.subcore_barrier()

            @pl.loop(0, NCHUNK)
            def _chunk(ci):
                base = sid * EP + ci * G
                pltpu.sync_copy(src_hbm.at[pl.ds(base, G)], srcv)
                pltpu.sync_copy(dst_hbm.at[pl.ds(base, G)], dstv)
                pltpu.sync_copy(w_hbm.at[pl.ds(base, G)], wv)
                off = b * M
                for j in range(G // 16):
                    gidx[pl.ds(j * 16, 16)] = srcv[pl.ds(j * 16, 16)] + off
                pltpu.async_copy(x_hbm.at[gidx], rows, gsem).wait()

                @pl.loop(0, G // 16)
                def _scale(g):
                    wvec = wv[pl.ds(g * 16, 16)]
                    for l in range(16):
                        we = wvec[l]
                        e = g * 16 + l
                        for j in range(F // 16):
                            rows[e, pl.ds(j * 16, 16)] = rows[e, pl.ds(j * 16, 16)] * we

                pltpu.sync_copy(rows, acc.at[dstv], add=True)

            plsc.subcore_barrier()
            pltpu.sync_copy(
                acc.at[pl.ds(sid * DR, DR)],
                y_hbm.at[pl.ds(b * M + sid * DR, DR)],
            )

            @pl.when(sid == NS - 1)
            def _dtail():
                pltpu.sync_copy(
                    acc.at[pl.ds(NS * DR, TAIL)],
                    y_hbm.at[pl.ds(b * M + NS * DR, TAIL)],
                )

    return run(xflat, src, dst, w)


def _mix_body(x_ref, a_ref, b_ref, wa_ref, wb_ref, wc_ref, bias_ref, o_ref):
    acc = jnp.dot(x_ref[...], wa_ref[...], preferred_element_type=jnp.float32)
    acc = acc + jnp.dot(a_ref[...], wb_ref[...], preferred_element_type=jnp.float32)
    acc = acc + jnp.dot(b_ref[...], wc_ref[...], preferred_element_type=jnp.float32)
    o_ref[...] = acc + bias_ref[...]


def _mix_tc(xflat, s1, s2, wa, wb, wc, bias2):
    TM = 1000
    grid = (B * M // TM,)
    row_spec = pl.BlockSpec((TM, F), lambda i: (i, 0))
    w_spec = pl.BlockSpec((F, F), lambda i: (0, 0))
    return pl.pallas_call(
        _mix_body,
        grid=grid,
        in_specs=[row_spec, row_spec, row_spec, w_spec, w_spec, w_spec,
                  pl.BlockSpec((1, F), lambda i: (0, 0))],
        out_specs=row_spec,
        out_shape=jax.ShapeDtypeStruct((B * M, F), jnp.float32),
    )(xflat, s1, s2, wa, wb, wc, bias2)


def kernel(x, edge_index, edge_weight, kernel, bias):
    xflat = x.reshape(B * M, F)
    src = edge_index[0]
    dst = edge_index[1]

    s1 = _spmm_sc(xflat, src, dst, edge_weight)
    s2 = _spmm_sc(s1, src, dst, edge_weight)

    w3 = kernel.reshape(F, K, -1)
    wa = w3[:, 0, :] - w3[:, 2, :]
    wb = w3[:, 1, :]
    wc = 2.0 * w3[:, 2, :]
    out = _mix_tc(xflat, s1, s2, wa, wb, wc, bias.reshape(1, -1))
    return out.reshape(B, M, -1)


# speedup vs baseline: 3.6026x; 2.7601x over previous
import functools

import jax
import jax.numpy as jnp
from jax import lax
from jax.experimental import pallas as pl
from jax.experimental.pallas import tpu as pltpu
from jax.experimental.pallas import tpu_sc as plsc

B, M, F, K, E = 8, 10000, 128, 3, 320000
NC, NS = 2, 16
EP = E // NS
G = 80
SG = 4000
CPS = SG // G
NSUP = EP // SG
DR = 624
TAIL = M - NS * DR
ZR = 52
BPC = B // NC


def _spmm_sc(xflat, src, dst, w):
    mesh = plsc.VectorSubcoreMesh(
        core_axis_name="c", subcore_axis_name="s", num_cores=NC, num_subcores=NS
    )

    @functools.partial(
        pl.kernel,
        out_type=jax.ShapeDtypeStruct((B * M, F), jnp.float32),
        mesh=mesh,
        scratch_types=[
            pltpu.VMEM_SHARED((M, F), jnp.float32),
            pltpu.VMEM((ZR, F), jnp.float32),
            pltpu.VMEM((SG,), jnp.int32),
            pltpu.VMEM((SG,), jnp.int32),
            pltpu.VMEM((SG,), jnp.float32),
            pltpu.VMEM((SG,), jnp.int32),
            pltpu.VMEM((G, F), jnp.float32),
            pltpu.VMEM((G, F), jnp.float32),
            pltpu.VMEM((G,), jnp.int32),
            pltpu.VMEM((G,), jnp.int32),
            pltpu.SemaphoreType.DMA,
            pltpu.SemaphoreType.DMA,
        ],
    )
    def run(x_hbm, src_hbm, dst_hbm, w_hbm, y_hbm,
            acc, zbuf, srcv, dstv, wv, gidx, rows0, rows1, d0, d1, sem0, sem1):
        cid = lax.axis_index("c")
        sid = lax.axis_index("s")
        ebase = sid * EP

        @pl.loop(0, ZR)
        def _zfill(r):
            for j in range(F // 16):
                zbuf[r, pl.ds(j * 16, 16)] = jnp.zeros((16,), jnp.float32)

        rows = (rows0, rows1)
        dbuf = (d0, d1)
        sems = (sem0, sem1)

        def gather(ci, p):
            pltpu.async_copy(x_hbm.at[gidx.at[pl.ds(ci * G, G)]], rows[p], sems[p])

        def process(ci, p):
            pltpu.make_async_copy(
                x_hbm.at[gidx.at[pl.ds(ci * G, G)]], rows[p], sems[p]
            ).wait()
            for j in range(G // 16):
                dbuf[p][pl.ds(j * 16, 16)] = dstv[pl.ds(ci * G + j * 16, 16)]

            @pl.loop(0, G // 16)
            def _scale(g):
                wvec = wv[pl.ds(ci * G + g * 16, 16)]
                for l in range(16):
                    we = wvec[l]
                    e = g * 16 + l
                    for j in range(F // 16):
                        rows[p][e, pl.ds(j * 16, 16)] = (
                            rows[p][e, pl.ds(j * 16, 16)] * we
                        )

            pltpu.sync_copy(rows[p], acc.at[dbuf[p]], add=True)

        @pl.loop(0, BPC)
        def _batch(bi):
            b = cid * BPC + bi
            off = b * M

            for z in range(DR // ZR):
                pltpu.sync_copy(zbuf, acc.at[pl.ds(sid * DR + z * ZR, ZR)])

            @pl.when(sid == NS - 1)
            def _ztail():
                pltpu.sync_copy(zbuf.at[pl.ds(0, TAIL)], acc.at[pl.ds(NS * DR, TAIL)])

            plsc.subcore_barrier()

            @pl.loop(0, NSUP)
            def _sup(sc):
                sbase = ebase + sc * SG
                pltpu.sync_copy(src_hbm.at[pl.ds(sbase, SG)], srcv)
                pltpu.sync_copy(dst_hbm.at[pl.ds(sbase, SG)], dstv)
                pltpu.sync_copy(w_hbm.at[pl.ds(sbase, SG)], wv)

                @pl.loop(0, SG // 16)
                def _rebase(i):
                    gidx[pl.ds(i * 16, 16)] = srcv[pl.ds(i * 16, 16)] + off

                gather(0, 0)

                @pl.loop(0, CPS, step=2)
                def _chunk(ci):
                    gather(ci + 1, 1)
                    process(ci, 0)

                    @pl.when(ci + 2 < CPS)
                    def _pre():
                        gather(ci + 2, 0)

                    process(ci + 1, 1)

            plsc.subcore_barrier()
            pltpu.sync_copy(
                acc.at[pl.ds(sid * DR, DR)],
                y_hbm.at[pl.ds(off + sid * DR, DR)],
            )

            @pl.when(sid == NS - 1)
            def _dtail():
                pltpu.sync_copy(
                    acc.at[pl.ds(NS * DR, TAIL)],
                    y_hbm.at[pl.ds(off + NS * DR, TAIL)],
                )

    return run(xflat, src, dst, w)


def _mix_body(x_ref, a_ref, b_ref, wa_ref, wb_ref, wc_ref, bias_ref, o_ref):
    acc = jnp.dot(x_ref[...], wa_ref[...], preferred_element_type=jnp.float32)
    acc = acc + jnp.dot(a_ref[...], wb_ref[...], preferred_element_type=jnp.float32)
    acc = acc + jnp.dot(b_ref[...], wc_ref[...], preferred_element_type=jnp.float32)
    o_ref[...] = acc + bias_ref[...]


def _mix_tc(xflat, s1, s2, wa, wb, wc, bias2):
    TM = 1000
    grid = (B * M // TM,)
    row_spec = pl.BlockSpec((TM, F), lambda i: (i, 0))
    w_spec = pl.BlockSpec((F, F), lambda i: (0, 0))
    return pl.pallas_call(
        _mix_body,
        grid=grid,
        in_specs=[row_spec, row_spec, row_spec, w_spec, w_spec, w_spec,
                  pl.BlockSpec((1, F), lambda i: (0, 0))],
        out_specs=row_spec,
        out_shape=jax.ShapeDtypeStruct((B * M, F), jnp.float32),
    )(xflat, s1, s2, wa, wb, wc, bias2)


def kernel(x, edge_index, edge_weight, kernel, bias):
    xflat = x.reshape(B * M, F)
    src = edge_index[0]
    dst = edge_index[1]

    s1 = _spmm_sc(xflat, src, dst, edge_weight)
    s2 = _spmm_sc(s1, src, dst, edge_weight)

    w3 = kernel.reshape(F, K, -1)
    wa = w3[:, 0, :] - w3[:, 2, :]
    wb = w3[:, 1, :]
    wc = 2.0 * w3[:, 2, :]
    out = _mix_tc(xflat, s1, s2, wa, wb, wc, bias.reshape(1, -1))
    return out.reshape(B, M, -1)


# speedup vs baseline: 3.9108x; 1.0855x over previous
import functools

import jax
import jax.numpy as jnp
from jax import lax
from jax.experimental import pallas as pl
from jax.experimental.pallas import tpu as pltpu
from jax.experimental.pallas import tpu_sc as plsc

B, M, F, K, E = 8, 10000, 128, 3, 320000
NC, NS = 2, 16
EP = E // NS
G = 80
SG = 2000
CPS = SG // G
NSUP = EP // SG
DR = 624
TAIL = M - NS * DR
ZR = 52
BPC = B // NC


def _spmm_sc(xflat, src, dst, w):
    mesh = plsc.VectorSubcoreMesh(
        core_axis_name="c", subcore_axis_name="s", num_cores=NC, num_subcores=NS
    )

    @functools.partial(
        pl.kernel,
        out_type=jax.ShapeDtypeStruct((B * M, F), jnp.float32),
        mesh=mesh,
        scratch_types=[
            pltpu.VMEM_SHARED((M, F), jnp.float32),
            pltpu.VMEM((ZR, F), jnp.float32),
            pltpu.VMEM((SG,), jnp.int32),
            pltpu.VMEM((SG,), jnp.int32),
            pltpu.VMEM((SG,), jnp.float32),
            pltpu.VMEM((SG,), jnp.int32),
            pltpu.VMEM((G, F), jnp.float32),
            pltpu.VMEM((G, F), jnp.float32),
            pltpu.VMEM((G, F), jnp.float32),
            pltpu.VMEM((G,), jnp.int32),
            pltpu.VMEM((G,), jnp.int32),
            pltpu.VMEM((G,), jnp.int32),
            pltpu.SemaphoreType.DMA,
            pltpu.SemaphoreType.DMA,
            pltpu.SemaphoreType.DMA,
            pltpu.SemaphoreType.DMA,
            pltpu.SemaphoreType.DMA,
            pltpu.SemaphoreType.DMA,
        ],
    )
    def run(x_hbm, src_hbm, dst_hbm, w_hbm, y_hbm,
            acc, zbuf, srcv, dstv, wv, gidx, rows0, rows1, rows2,
            d0, d1, d2, gsem0, gsem1, gsem2, ssem0, ssem1, ssem2):
        cid = lax.axis_index("c")
        sid = lax.axis_index("s")
        ebase = sid * EP

        @pl.loop(0, ZR)
        def _zfill(r):
            for j in range(F // 16):
                zbuf[r, pl.ds(j * 16, 16)] = jnp.zeros((16,), jnp.float32)

        rows = (rows0, rows1, rows2)
        dbuf = (d0, d1, d2)
        gsems = (gsem0, gsem1, gsem2)
        ssems = (ssem0, ssem1, ssem2)

        def gather(ci, p):
            pltpu.async_copy(x_hbm.at[gidx.at[pl.ds(ci * G, G)]], rows[p], gsems[p])

        def wait_scatter(p):
            pltpu.make_async_copy(rows[p], acc.at[dbuf[p]], ssems[p]).wait()

        def process(ci, p):
            pltpu.make_async_copy(
                x_hbm.at[gidx.at[pl.ds(ci * G, G)]], rows[p], gsems[p]
            ).wait()
            for j in range(G // 16):
                dbuf[p][pl.ds(j * 16, 16)] = dstv[pl.ds(ci * G + j * 16, 16)]

            @pl.loop(0, G // 16)
            def _scale(g):
                wvec = wv[pl.ds(ci * G + g * 16, 16)]
                for l in range(16):
                    we = wvec[l]
                    e = g * 16 + l
                    for j in range(F // 16):
                        rows[p][e, pl.ds(j * 16, 16)] = (
                            rows[p][e, pl.ds(j * 16, 16)] * we
                        )

            pltpu.async_copy(rows[p], acc.at[dbuf[p]], ssems[p], add=True)

        @pl.loop(0, BPC)
        def _batch(bi):
            b = cid * BPC + bi
            off = b * M

            for z in range(DR // ZR):
                pltpu.sync_copy(zbuf, acc.at[pl.ds(sid * DR + z * ZR, ZR)])

            @pl.when(sid == NS - 1)
            def _ztail():
                pltpu.sync_copy(zbuf.at[pl.ds(0, TAIL)], acc.at[pl.ds(NS * DR, TAIL)])

            plsc.subcore_barrier()

            @pl.loop(0, NSUP)
            def _sup(sc):
                sbase = ebase + sc * SG
                pltpu.sync_copy(src_hbm.at[pl.ds(sbase, SG)], srcv)
                pltpu.sync_copy(dst_hbm.at[pl.ds(sbase, SG)], dstv)
                pltpu.sync_copy(w_hbm.at[pl.ds(sbase, SG)], wv)

                @pl.loop(0, SG // 16)
                def _rebase(i):
                    gidx[pl.ds(i * 16, 16)] = srcv[pl.ds(i * 16, 16)] + off

                gather(0, 0)

                @pl.loop(0, CPS - 1, step=3)
                def _chunk(ci):
                    for r in range(3):
                        c = ci + r
                        q = (r + 1) % 3

                        @pl.when(c >= 2)
                        def _ws():
                            wait_scatter(q)

                        gather(c + 1, q)
                        process(c, r)

                wait_scatter(1)
                process(CPS - 1, 0)
                wait_scatter(2)
                wait_scatter(0)

            plsc.subcore_barrier()
            pltpu.sync_copy(
                acc.at[pl.ds(sid * DR, DR)],
                y_hbm.at[pl.ds(off + sid * DR, DR)],
            )

            @pl.when(sid == NS - 1)
            def _dtail():
                pltpu.sync_copy(
                    acc.at[pl.ds(NS * DR, TAIL)],
                    y_hbm.at[pl.ds(off + NS * DR, TAIL)],
                )

    return run(xflat, src, dst, w)


def _mix_body(x_ref, a_ref, b_ref, wa_ref, wb_ref, wc_ref, bias_ref, o_ref):
    acc = jnp.dot(x_ref[...], wa_ref[...], preferred_element_type=jnp.float32)
    acc = acc + jnp.dot(a_ref[...], wb_ref[...], preferred_element_type=jnp.float32)
    acc = acc + jnp.dot(b_ref[...], wc_ref[...], preferred_element_type=jnp.float32)
    o_ref[...] = acc + bias_ref[...]


def _mix_tc(xflat, s1, s2, wa, wb, wc, bias2):
    TM = 1000
    grid = (B * M // TM,)
    row_spec = pl.BlockSpec((TM, F), lambda i: (i, 0))
    w_spec = pl.BlockSpec((F, F), lambda i: (0, 0))
    return pl.pallas_call(
        _mix_body,
        grid=grid,
        in_specs=[row_spec, row_spec, row_spec, w_spec, w_spec, w_spec,
                  pl.BlockSpec((1, F), lambda i: (0, 0))],
        out_specs=row_spec,
        out_shape=jax.ShapeDtypeStruct((B * M, F), jnp.float32),
    )(xflat, s1, s2, wa, wb, wc, bias2)


def kernel(x, edge_index, edge_weight, kernel, bias):
    xflat = x.reshape(B * M, F)
    src = edge_index[0]
    dst = edge_index[1]

    s1 = _spmm_sc(xflat, src, dst, edge_weight)
    s2 = _spmm_sc(s1, src, dst, edge_weight)

    w3 = kernel.reshape(F, K, -1)
    wa = w3[:, 0, :] - w3[:, 2, :]
    wb = w3[:, 1, :]
    wc = 2.0 * w3[:, 2, :]
    out = _mix_tc(xflat, s1, s2, wa, wb, wc, bias.reshape(1, -1))
    return out.reshape(B, M, -1)
